# baseline (speedup 1.0000x reference)
import jax
import jax.numpy as jnp
from jax.experimental import pallas as pl
from jax.experimental.pallas import tpu as pltpu

NC = 1000
B0 = 4096
B1 = 26
LW = 2048
NLB = B0 // LW
NSTEP = B1 * NLB
NBUF = 6


def _onehot_body(x_ref, out_ref, scratch, sems):
    i = pl.program_id(0)
    j = i // NLB
    h = jax.lax.rem(i, NLB)
    b = jax.lax.rem(i, NBUF)

    def mkcopy(bb, jj, hh):
        return pltpu.make_async_copy(
            scratch.at[bb],
            out_ref.at[pl.ds(jj, 1), :, pl.ds(hh * LW, LW)],
            sems.at[bb],
        )

    @pl.when(i >= NBUF)
    def _wait_prev():
        mkcopy(b, j, h).wait()

    iota = jax.lax.broadcasted_iota(jnp.int32, (1, NC, LW), 1)
    xv = x_ref[:, pl.ds(j, 1), pl.ds(h * LW, LW)]
    scratch[b] = (iota == xv).astype(jnp.float32)
    mkcopy(b, j, h).start()

    @pl.when(i == NSTEP - 1)
    def _drain():
        for bb in range(NBUF):
            mkcopy(bb, j, h).wait()


def kernel(x):
    xt = x.astype(jnp.int32).T.reshape(1, B1, B0)
    out_t = pl.pallas_call(
        _onehot_body,
        grid=(NSTEP,),
        in_specs=[pl.BlockSpec((1, B1, B0), lambda j: (0, 0, 0))],
        out_specs=pl.BlockSpec(memory_space=pl.ANY),
        out_shape=jax.ShapeDtypeStruct((B1, NC, B0), jnp.float32),
        scratch_shapes=[
            pltpu.VMEM((NBUF, 1, NC, LW), jnp.float32),
            pltpu.SemaphoreType.DMA((NBUF,)),
        ],
    )(xt)
    return jnp.transpose(out_t, (2, 0, 1))


# speedup vs baseline: 1.0664x; 1.0664x over previous
import jax
import jax.numpy as jnp
from jax.experimental import pallas as pl
from jax.experimental.pallas import tpu as pltpu

NC = 1000
B0 = 4096
B1 = 26
LW = 2048
NLB = B0 // LW
NSTEP = B1 * NLB
NBUF = 3


def _onehot_body(x_ref, out_ref, scratch, sems):
    i = pl.program_id(0)
    j = i // NLB
    h = jax.lax.rem(i, NLB)
    b = jax.lax.rem(i, NBUF)

    def mkcopy(bb, jj, hh):
        return pltpu.make_async_copy(
            scratch.at[bb],
            out_ref.at[pl.ds(jj, 1), :, pl.ds(hh * LW, LW)],
            sems.at[bb],
        )

    @pl.when(i >= NBUF)
    def _wait_prev():
        mkcopy(b, j, h).wait()

    iota = jax.lax.broadcasted_iota(jnp.int32, (1, NC, LW), 1)
    xv = x_ref[:, pl.ds(j, 1), pl.ds(h * LW, LW)]
    scratch[b] = (iota == xv).astype(jnp.float32)
    mkcopy(b, j, h).start()

    @pl.when(i == NSTEP - 1)
    def _drain():
        for bb in range(NBUF):
            mkcopy(bb, j, h).wait()


def kernel(x):
    xt = x.astype(jnp.int32).T.reshape(1, B1, B0)
    out_t = pl.pallas_call(
        _onehot_body,
        grid=(NSTEP,),
        in_specs=[pl.BlockSpec((1, B1, B0), lambda j: (0, 0, 0))],
        out_specs=pl.BlockSpec(memory_space=pl.ANY),
        out_shape=jax.ShapeDtypeStruct((B1, NC, B0), jnp.float32),
        scratch_shapes=[
            pltpu.VMEM((NBUF, 1, NC, LW), jnp.float32),
            pltpu.SemaphoreType.DMA((NBUF,)),
        ],
    )(xt)
    return jnp.transpose(out_t, (2, 0, 1))
